# baseline (speedup 1.0000x reference)
import functools

import jax
import jax.numpy as jnp
from jax import lax
from jax.experimental import pallas as pl
from jax.experimental.pallas import tpu as pltpu
from jax.experimental.pallas import tpu_sc as plsc

V = 10000
E = 64
BS = 16384
XMAX = 100.0
ALPHA = 0.75

NC = 2
NS = 16
L = 16
NW = NC * NS
BPW = BS // NW
HALF = BPW // 2
CH = 128
NWAVE = HALF // L


def _sc_body(word_h, ctx_h, tab_h, bw_h, bc_h, comat_h, pre_h, co_h,
             word_v, ctx_v, bw_v, bc_v, wrows, crows, co_v, pre_v,
             tile_a, tile_b, sem, semr):
    wid = lax.axis_index("s") * NC + lax.axis_index("c")
    base = wid * BPW

    pltpu.sync_copy(word_h.at[pl.ds(base, BPW)], word_v)
    pltpu.sync_copy(ctx_h.at[pl.ds(base, BPW)], ctx_v)
    pltpu.sync_copy(bw_h, bw_v)
    pltpu.sync_copy(bc_h, bc_v)

    lane = lax.iota(jnp.int32, L)

    for half in range(2):
        hb = half * HALF
        row_copies = []
        for j in range(HALF // CH):
            sl = pl.ds(hb + j * CH, CH)
            dsl = pl.ds(j * CH, CH)
            row_copies.append(
                pltpu.async_copy(tab_h.at[word_v.at[sl]], wrows.at[dsl], semr))
            row_copies.append(
                pltpu.async_copy(tab_h.at[ctx_v.at[sl]], crows.at[dsl], semr))

        def wave_pair(p, carry):
            def fire(buf, w):
                ws = word_v[pl.ds(hb + w * L, L)]
                cs = ctx_v[pl.ds(hb + w * L, L)]
                cps = []
                for j in range(L):
                    rb = pl.multiple_of((ws[j] >> 3) << 3, 8)
                    cb = pl.multiple_of((cs[j] >> 7) << 7, 128)
                    cps.append(pltpu.async_copy(
                        comat_h.at[pl.ds(rb, 8), pl.ds(cb, 128)],
                        buf.at[j], sem))
                return ws, cs, cps

            def extract(buf, ws, cs, cps, w):
                for cp in cps:
                    cp.wait()
                co_v[pl.ds(hb + w * L, L)] = plsc.load_gather(
                    buf, [lane, ws & 7, cs & 127])

            wa = p * 2
            wsa, csa, cpsa = fire(tile_a, wa)
            wsb, csb, cpsb = fire(tile_b, wa + 1)
            extract(tile_a, wsa, csa, cpsa, wa)
            extract(tile_b, wsb, csb, cpsb, wa + 1)
            return carry

        lax.fori_loop(0, NWAVE // 2, wave_pair, 0)

        for cp in row_copies:
            cp.wait()

        def group(g, carry):
            rid = g * L + lane
            acc = jnp.zeros((L,), jnp.float32)
            for e in range(E):
                ev = jnp.full((L,), e, jnp.int32)
                wv = plsc.load_gather(wrows, [rid, ev])
                cv = plsc.load_gather(crows, [rid, ev + E])
                acc = acc + wv * cv
            sl = pl.ds(hb + g * L, L)
            bwg = plsc.load_gather(bw_v, [word_v[sl]])
            bcg = plsc.load_gather(bc_v, [ctx_v[sl]])
            pre_v[sl] = acc + bwg + bcg
            return carry

        lax.fori_loop(0, HALF // L, group, 0)

    pltpu.sync_copy(pre_v, pre_h.at[pl.ds(base, BPW)])
    pltpu.sync_copy(co_v, co_h.at[pl.ds(base, BPW)])


_sc_gather = functools.partial(
    pl.kernel,
    out_type=(jax.ShapeDtypeStruct((BS,), jnp.float32),
              jax.ShapeDtypeStruct((BS,), jnp.float32)),
    mesh=plsc.VectorSubcoreMesh(core_axis_name="c", subcore_axis_name="s",
                                num_cores=NC, num_subcores=NS),
    compiler_params=pltpu.CompilerParams(needs_layout_passes=False),
    scratch_types=[
        pltpu.VMEM((BPW,), jnp.int32),
        pltpu.VMEM((BPW,), jnp.int32),
        pltpu.VMEM((V,), jnp.float32),
        pltpu.VMEM((V,), jnp.float32),
        pltpu.VMEM((HALF, 2 * E), jnp.float32),
        pltpu.VMEM((HALF, 2 * E), jnp.float32),
        pltpu.VMEM((BPW,), jnp.float32),
        pltpu.VMEM((BPW,), jnp.float32),
        pltpu.VMEM((L, 8, 128), jnp.float32),
        pltpu.VMEM((L, 8, 128), jnp.float32),
        pltpu.SemaphoreType.DMA,
        pltpu.SemaphoreType.DMA,
    ],
)(_sc_body)


def _tc_body(pre_ref, co_ref, out_ref):
    co = co_ref[...]
    pre = pre_ref[...]
    w = jnp.where(co < XMAX, (co * (1.0 / XMAX)) ** ALPHA,
                  jnp.ones_like(co))
    d = pre - jnp.log(co)
    out_ref[0, 0] = jnp.sum(d * d * w)


_tc_loss = pl.pallas_call(
    _tc_body,
    out_shape=jax.ShapeDtypeStruct((1, 1), jnp.float32),
    out_specs=pl.BlockSpec(memory_space=pltpu.SMEM),
)


def kernel(word, context, Wword, Wctx, bword, bctx, comat):
    word = word.astype(jnp.int32)
    context = context.astype(jnp.int32)
    table = jnp.concatenate([Wword, Wctx], axis=1)
    pre, co = _sc_gather(word, context, table,
                         bword.reshape(-1), bctx.reshape(-1), comat)
    out = _tc_loss(pre.reshape(BS // 128, 128), co.reshape(BS // 128, 128))
    return out[0, 0]


# speedup vs baseline: 1.0041x; 1.0041x over previous
import functools

import jax
import jax.numpy as jnp
from jax import lax
from jax.experimental import pallas as pl
from jax.experimental.pallas import tpu as pltpu
from jax.experimental.pallas import tpu_sc as plsc

V = 10000
E = 64
BS = 16384
XMAX = 100.0
ALPHA = 0.75

NC = 2
NS = 16
L = 16
NW = NC * NS
BPW = BS // NW
HALF = BPW // 2
CH = 128
NWAVE = HALF // L

_LN2 = 0.6931471805599453
_LNXMAX = 4.605170185988092


def _vlog(x):
    bits = plsc.bitcast(x, jnp.int32)
    e = ((bits >> 23) & 255) - 127
    m = plsc.bitcast((bits & 0x007FFFFF) | 0x3F800000, jnp.float32)
    t = (m - 1.0) / (m + 1.0)
    t2 = t * t
    lnm = 2.0 * t * (1.0 + t2 * (1.0 / 3 + t2 * (1.0 / 5 + t2 * (1.0 / 7))))
    return e.astype(jnp.float32) * _LN2 + lnm


def _sc_body(word_h, ctx_h, tab_h, bw_h, bc_h, comat_h, out_h,
             word_v, ctx_v, bw_v, bc_v, wrows, crows, co_v,
             tile_a, tile_b, sem, semr):
    wid = lax.axis_index("s") * NC + lax.axis_index("c")
    base = wid * BPW

    pltpu.sync_copy(word_h.at[pl.ds(base, BPW)], word_v)
    pltpu.sync_copy(ctx_h.at[pl.ds(base, BPW)], ctx_v)
    pltpu.sync_copy(bw_h, bw_v)
    pltpu.sync_copy(bc_h, bc_v)

    lane = lax.iota(jnp.int32, L)
    acc = jnp.zeros((L,), jnp.float32)

    for half in range(2):
        hb = half * HALF
        row_copies = []
        for j in range(HALF // CH):
            sl = pl.ds(hb + j * CH, CH)
            dsl = pl.ds(j * CH, CH)
            row_copies.append(
                pltpu.async_copy(tab_h.at[word_v.at[sl]], wrows.at[dsl], semr))
            row_copies.append(
                pltpu.async_copy(tab_h.at[ctx_v.at[sl]], crows.at[dsl], semr))

        def wave_pair(p, carry):
            def fire(buf, w):
                ws = word_v[pl.ds(hb + w * L, L)]
                cs = ctx_v[pl.ds(hb + w * L, L)]
                cps = []
                for j in range(L):
                    rb = pl.multiple_of((ws[j] >> 3) << 3, 8)
                    cb = pl.multiple_of((cs[j] >> 7) << 7, 128)
                    cps.append(pltpu.async_copy(
                        comat_h.at[pl.ds(rb, 8), pl.ds(cb, 128)],
                        buf.at[j], sem))
                return ws, cs, cps

            def extract(buf, ws, cs, cps, w):
                for cp in cps:
                    cp.wait()
                co_v[pl.ds(hb + w * L, L)] = plsc.load_gather(
                    buf, [lane, ws & 7, cs & 127])

            wa = p * 2
            wsa, csa, cpsa = fire(tile_a, wa)
            wsb, csb, cpsb = fire(tile_b, wa + 1)
            extract(tile_a, wsa, csa, cpsa, wa)
            extract(tile_b, wsb, csb, cpsb, wa + 1)
            return carry

        lax.fori_loop(0, NWAVE // 2, wave_pair, 0)

        for cp in row_copies:
            cp.wait()

        def group(g, acc_in):
            rid = g * L + lane
            dot = jnp.zeros((L,), jnp.float32)
            for e in range(E):
                ev = jnp.full((L,), e, jnp.int32)
                wv = plsc.load_gather(wrows, [rid, ev])
                cv = plsc.load_gather(crows, [rid, ev + E])
                dot = dot + wv * cv
            sl = pl.ds(hb + g * L, L)
            bwg = plsc.load_gather(bw_v, [word_v[sl]])
            bcg = plsc.load_gather(bc_v, [ctx_v[sl]])
            co = co_v[sl]
            lnco = _vlog(co)
            wgt = jnp.where(co < XMAX,
                            jnp.exp(ALPHA * (lnco - _LNXMAX)),
                            jnp.ones_like(co))
            d = dot + bwg + bcg - lnco
            return acc_in + d * d * wgt

        acc = lax.fori_loop(0, HALF // L, group, acc)

    co_v[pl.ds(0, L)] = acc
    pltpu.sync_copy(co_v.at[pl.ds(0, L)], out_h.at[pl.ds(wid * L, L)])


_sc_loss = functools.partial(
    pl.kernel,
    out_type=jax.ShapeDtypeStruct((NW * L,), jnp.float32),
    mesh=plsc.VectorSubcoreMesh(core_axis_name="c", subcore_axis_name="s",
                                num_cores=NC, num_subcores=NS),
    compiler_params=pltpu.CompilerParams(needs_layout_passes=False),
    scratch_types=[
        pltpu.VMEM((BPW,), jnp.int32),
        pltpu.VMEM((BPW,), jnp.int32),
        pltpu.VMEM((V,), jnp.float32),
        pltpu.VMEM((V,), jnp.float32),
        pltpu.VMEM((HALF, 2 * E), jnp.float32),
        pltpu.VMEM((HALF, 2 * E), jnp.float32),
        pltpu.VMEM((BPW,), jnp.float32),
        pltpu.VMEM((L, 8, 128), jnp.float32),
        pltpu.VMEM((L, 8, 128), jnp.float32),
        pltpu.SemaphoreType.DMA,
        pltpu.SemaphoreType.DMA,
    ],
)(_sc_body)


def kernel(word, context, Wword, Wctx, bword, bctx, comat):
    word = word.astype(jnp.int32)
    context = context.astype(jnp.int32)
    table = jnp.concatenate([Wword, Wctx], axis=1)
    parts = _sc_loss(word, context, table,
                     bword.reshape(-1), bctx.reshape(-1), comat)
    return jnp.sum(parts)
